# baseline (speedup 1.0000x reference)
import functools

import jax
import jax.numpy as jnp
from jax import lax
from jax.experimental import pallas as pl
from jax.experimental.pallas import tpu as pltpu
from jax.experimental.pallas import tpu_sc as plsc

_D = 256
_DD = 2 * _D
_L = 16
_SL = 4
_CHUNK = 64


def _bf16_bits(x):
    u = lax.bitcast_convert_type(x, jnp.uint32)
    return (u + 0x7FFF + ((u >> 16) & 1)) >> 16


def _normalize_body(s_ref, t_ref, o_ref):
    h = _D // 2
    for ref, col in ((s_ref, 0), (t_ref, h)):
        x = ref[...] * 2.0
        m = jnp.max(x, axis=1, keepdims=True)
        e = jnp.exp(x - m)
        p = e / jnp.sum(e, axis=1, keepdims=True)
        n = jnp.sqrt(jnp.sum(p * p, axis=1, keepdims=True))
        f = p / jnp.maximum(n, 1e-12)
        packed = _bf16_bits(f[:, :h]) | (_bf16_bits(f[:, h:]) << 16)
        o_ref[:, col:col + h] = lax.bitcast_convert_type(packed, jnp.int32)


def _build_table(student_out, teacher_out):
    n = student_out.shape[0]
    block = 2000
    grid = n // block
    return pl.pallas_call(
        _normalize_body,
        grid=(grid,),
        in_specs=[
            pl.BlockSpec((block, _D), lambda i: (i, 0)),
            pl.BlockSpec((block, _D), lambda i: (i, 0)),
        ],
        out_specs=pl.BlockSpec((block, _DD // 2), lambda i: (i, 0)),
        out_shape=jax.ShapeDtypeStruct((n, _DD // 2), jnp.int32),
    )(student_out, teacher_out)


def _edge_loss_body(num_edges, table, edges, outh,
                    idx_s, idx_d, rows_s0, rows_d0, rows_s1, rows_d1,
                    rows_s2, rows_d2, pmat, out_v,
                    sem_s0, sem_d0, sem_s1, sem_d1, sem_s2, sem_d2):
    info = plsc.get_sparse_core_info()
    nc = info.num_cores
    nw = nc * info.num_subcores
    epw = num_edges // nw
    nfull = epw // _CHUNK
    tail = epw - nfull * _CHUNK
    wid = lax.axis_index("s") * nc + lax.axis_index("c")
    base = wid * epw

    zeros = jnp.zeros((_L,), jnp.float32)
    iota = lax.iota(jnp.int32, _L)
    bufs = ((rows_s0, rows_d0, sem_s0, sem_d0),
            (rows_s1, rows_d1, sem_s1, sem_d1),
            (rows_s2, rows_d2, sem_s2, sem_d2))

    pltpu.sync_copy(edges.at[pl.ds(base, epw)], idx_s)
    pltpu.sync_copy(edges.at[pl.ds(num_edges + base, epw)], idx_d)

    def start(c, b, n_rows):
        rs, rd, ss, sd = bufs[b]
        pltpu.async_copy(table.at[idx_s.at[pl.ds(c * _CHUNK, n_rows)]],
                         rs.at[pl.ds(0, n_rows)], ss)
        pltpu.async_copy(table.at[idx_d.at[pl.ds(c * _CHUNK, n_rows)]],
                         rd.at[pl.ds(0, n_rows)], sd)

    def wait(b, n_rows):
        rs, rd, ss, sd = bufs[b]
        pltpu.make_async_copy(table.at[idx_s.at[pl.ds(0, n_rows)]],
                              rs.at[pl.ds(0, n_rows)], ss).wait()
        pltpu.make_async_copy(table.at[idx_d.at[pl.ds(0, n_rows)]],
                              rd.at[pl.ds(0, n_rows)], sd).wait()

    def compute(b, ngroups, lv):
        rs, rd, _, _ = bufs[b]

        def group_body(g, lv):
            @plsc.parallel_loop(0, _L, 1, unroll=2)
            def edge_body(e16):
                e = g * _L + e16
                acc = zeros
                for j in range(_DD // (2 * _L)):
                    sv = plsc.bitcast(rs[e, pl.ds(j * _L, _L)], jnp.bfloat16)
                    dv = plsc.bitcast(rd[e, pl.ds(j * _L, _L)], jnp.bfloat16)
                    pa, pb = plsc.unpack(
                        sv * dv, format=plsc.PackFormat.INTERLEAVED,
                        preferred_element_type=jnp.float32)
                    prod = pa + pb
                    if j < _D // (2 * _L):
                        acc = acc + prod
                    else:
                        acc = acc - prod
                pmat[e16, :] = acc
            diff = zeros
            for c in range(_L):
                diff = diff + plsc.load_gather(
                    pmat, [iota, jnp.full((_L,), c, jnp.int32)])
            return lv + diff * diff

        return lax.fori_loop(0, ngroups, group_body, lv)

    start(0, 0, _CHUNK)
    start(1, 1, _CHUNK)

    def triple_body(i, lv):
        for u in range(3):
            c = 3 * i + u

            @pl.when(c + 2 < nfull)
            def _():
                start(c + 2, (u + 2) % 3, _CHUNK)
            wait(u, _CHUNK)
            lv = compute(u, _CHUNK // _L, lv)
        return lv

    loss_vec = lax.fori_loop(0, nfull // 3, triple_body, zeros)

    start(nfull, nfull % 3, tail)
    izeros = jnp.zeros((_L,), jnp.int32)
    rs_t, rd_t, _, _ = bufs[nfull % 3]
    for r in range(tail, _L):
        for k in range(_DD // (2 * _L)):
            rs_t[r, pl.ds(k * _L, _L)] = izeros
            rd_t[r, pl.ds(k * _L, _L)] = izeros
    wait(nfull % 3, tail)
    loss_vec = compute(nfull % 3, 1, loss_vec)

    out_v[...] = loss_vec * (1.0 / num_edges)
    pltpu.sync_copy(out_v, outh.at[wid])


def _edge_loss(table, edges):
    num_edges = edges.shape[0] // 2
    info = plsc.get_sparse_core_info()
    nw = info.num_cores * info.num_subcores
    epw = num_edges // nw
    mesh = plsc.VectorSubcoreMesh(core_axis_name="c", subcore_axis_name="s")
    fn = pl.kernel(
        functools.partial(_edge_loss_body, num_edges),
        out_type=jax.ShapeDtypeStruct((nw, _L), jnp.float32),
        mesh=mesh,
        compiler_params=pltpu.CompilerParams(needs_layout_passes=False),
        scratch_types=[
            pltpu.VMEM((epw,), jnp.int32),
            pltpu.VMEM((epw,), jnp.int32),
            pltpu.VMEM((_CHUNK, _DD // 2), jnp.int32),
            pltpu.VMEM((_CHUNK, _DD // 2), jnp.int32),
            pltpu.VMEM((_CHUNK, _DD // 2), jnp.int32),
            pltpu.VMEM((_CHUNK, _DD // 2), jnp.int32),
            pltpu.VMEM((_CHUNK, _DD // 2), jnp.int32),
            pltpu.VMEM((_CHUNK, _DD // 2), jnp.int32),
            pltpu.VMEM((_L, _L), jnp.float32),
            pltpu.VMEM((_L,), jnp.float32),
            pltpu.SemaphoreType.DMA,
            pltpu.SemaphoreType.DMA,
            pltpu.SemaphoreType.DMA,
            pltpu.SemaphoreType.DMA,
            pltpu.SemaphoreType.DMA,
            pltpu.SemaphoreType.DMA,
        ],
    )
    return fn(table, edges)


def kernel(student_out, teacher_out, edge_index):
    table = _build_table(student_out, teacher_out)
    partials = _edge_loss(table, edge_index.reshape(-1))
    return jnp.sum(partials)


# speedup vs baseline: 1.0593x; 1.0593x over previous
import functools

import jax
import jax.numpy as jnp
from jax import lax
from jax.experimental import pallas as pl
from jax.experimental.pallas import tpu as pltpu
from jax.experimental.pallas import tpu_sc as plsc

_D = 256
_DD = 2 * _D
_L = 16
_SL = 4
_CHUNK = 64


def _bf16_bits(x):
    u = lax.bitcast_convert_type(x, jnp.uint32)
    return (u + 0x7FFF + ((u >> 16) & 1)) >> 16


def _normalize_body(s_ref, t_ref, o_ref):
    h = _D // 2
    for ref, col in ((s_ref, 0), (t_ref, h)):
        x = ref[...] * 2.0
        m = jnp.max(x, axis=1, keepdims=True)
        e = jnp.exp(x - m)
        p = e / jnp.sum(e, axis=1, keepdims=True)
        n = jnp.sqrt(jnp.sum(p * p, axis=1, keepdims=True))
        f = p / jnp.maximum(n, 1e-12)
        packed = _bf16_bits(f[:, :h]) | (_bf16_bits(f[:, h:]) << 16)
        o_ref[:, col:col + h] = lax.bitcast_convert_type(packed, jnp.int32)


def _build_table(student_out, teacher_out):
    n = student_out.shape[0]
    block = 2000
    grid = n // block
    return pl.pallas_call(
        _normalize_body,
        grid=(grid,),
        in_specs=[
            pl.BlockSpec((block, _D), lambda i: (i, 0)),
            pl.BlockSpec((block, _D), lambda i: (i, 0)),
        ],
        out_specs=pl.BlockSpec((block, _DD // 2), lambda i: (i, 0)),
        out_shape=jax.ShapeDtypeStruct((n, _DD // 2), jnp.int32),
    )(student_out, teacher_out)


def _edge_loss_body(num_edges, table, edges, outh,
                    idx_s, idx_d, rows_s0, rows_d0, rows_s1, rows_d1,
                    rows_s2, rows_d2, pmat, out_v,
                    sem_s0, sem_d0, sem_s1, sem_d1, sem_s2, sem_d2):
    info = plsc.get_sparse_core_info()
    nc = info.num_cores
    nw = nc * info.num_subcores
    epw = num_edges // nw
    nfull = epw // _CHUNK
    tail = epw - nfull * _CHUNK
    wid = lax.axis_index("s") * nc + lax.axis_index("c")
    base = wid * epw

    zeros = jnp.zeros((_L,), jnp.float32)
    iota = lax.iota(jnp.int32, _L)
    bufs = ((rows_s0, rows_d0, sem_s0, sem_d0),
            (rows_s1, rows_d1, sem_s1, sem_d1),
            (rows_s2, rows_d2, sem_s2, sem_d2))

    pltpu.sync_copy(edges.at[pl.ds(base, epw)], idx_s)
    pltpu.sync_copy(edges.at[pl.ds(num_edges + base, epw)], idx_d)

    def start(c, b, n_rows):
        rs, rd, ss, sd = bufs[b]
        pltpu.async_copy(table.at[idx_s.at[pl.ds(c * _CHUNK, n_rows)]],
                         rs.at[pl.ds(0, n_rows)], ss)
        pltpu.async_copy(table.at[idx_d.at[pl.ds(c * _CHUNK, n_rows)]],
                         rd.at[pl.ds(0, n_rows)], sd)

    def wait(b, n_rows):
        rs, rd, ss, sd = bufs[b]
        pltpu.make_async_copy(table.at[idx_s.at[pl.ds(0, n_rows)]],
                              rs.at[pl.ds(0, n_rows)], ss).wait()
        pltpu.make_async_copy(table.at[idx_d.at[pl.ds(0, n_rows)]],
                              rd.at[pl.ds(0, n_rows)], sd).wait()

    def compute(b, ngroups, lv):
        rs, rd, _, _ = bufs[b]

        def group_body(g, lv):
            @plsc.parallel_loop(0, _L, 1, unroll=2)
            def edge_body(e16):
                e = g * _L + e16
                acc = zeros
                for j in range(_DD // (2 * _L)):
                    sv = plsc.bitcast(rs[e, pl.ds(j * _L, _L)], jnp.bfloat16)
                    dv = plsc.bitcast(rd[e, pl.ds(j * _L, _L)], jnp.bfloat16)
                    pa, pb = plsc.unpack(
                        sv * dv, format=plsc.PackFormat.INTERLEAVED,
                        preferred_element_type=jnp.float32)
                    prod = pa + pb
                    if j < _D // (2 * _L):
                        acc = acc + prod
                    else:
                        acc = acc - prod
                pmat[e16, :] = acc
            diff = zeros
            for c in range(_L):
                diff = diff + plsc.load_gather(
                    pmat, [iota, jnp.full((_L,), c, jnp.int32)])
            return lv + diff * diff

        return lax.fori_loop(0, ngroups, group_body, lv)

    start(0, 0, _CHUNK)
    start(1, 1, _CHUNK)

    def triple_body(i, lv):
        for u in range(3):
            c = 3 * i + u

            @pl.when(c + 2 < nfull)
            def _():
                start(c + 2, (u + 2) % 3, _CHUNK)
            wait(u, _CHUNK)
            lv = compute(u, _CHUNK // _L, lv)
        return lv

    loss_vec = lax.fori_loop(0, nfull // 3, triple_body, zeros)

    start(nfull, nfull % 3, tail)
    izeros = jnp.zeros((_L,), jnp.int32)
    rs_t, rd_t, _, _ = bufs[nfull % 3]
    for r in range(tail, _L):
        for k in range(_DD // (2 * _L)):
            rs_t[r, pl.ds(k * _L, _L)] = izeros
            rd_t[r, pl.ds(k * _L, _L)] = izeros
    wait(nfull % 3, tail)
    loss_vec = compute(nfull % 3, 1, loss_vec)

    out_v[...] = loss_vec * (1.0 / num_edges)
    pltpu.sync_copy(out_v, outh.at[wid])


def _edge_loss(table, edges):
    num_edges = edges.shape[0] // 2
    info = plsc.get_sparse_core_info()
    nw = info.num_cores * info.num_subcores
    epw = num_edges // nw
    mesh = plsc.VectorSubcoreMesh(core_axis_name="c", subcore_axis_name="s")
    fn = pl.kernel(
        functools.partial(_edge_loss_body, num_edges),
        out_type=jax.ShapeDtypeStruct((nw, _L), jnp.float32),
        mesh=mesh,
        compiler_params=pltpu.CompilerParams(needs_layout_passes=False, use_tc_tiling_on_sc=False),
        scratch_types=[
            pltpu.VMEM((epw,), jnp.int32),
            pltpu.VMEM((epw,), jnp.int32),
            pltpu.VMEM((_CHUNK, _DD // 2), jnp.int32),
            pltpu.VMEM((_CHUNK, _DD // 2), jnp.int32),
            pltpu.VMEM((_CHUNK, _DD // 2), jnp.int32),
            pltpu.VMEM((_CHUNK, _DD // 2), jnp.int32),
            pltpu.VMEM((_CHUNK, _DD // 2), jnp.int32),
            pltpu.VMEM((_CHUNK, _DD // 2), jnp.int32),
            pltpu.VMEM((_L, _L), jnp.float32),
            pltpu.VMEM((_L,), jnp.float32),
            pltpu.SemaphoreType.DMA,
            pltpu.SemaphoreType.DMA,
            pltpu.SemaphoreType.DMA,
            pltpu.SemaphoreType.DMA,
            pltpu.SemaphoreType.DMA,
            pltpu.SemaphoreType.DMA,
        ],
    )
    return fn(table, edges)


def kernel(student_out, teacher_out, edge_index):
    table = _build_table(student_out, teacher_out)
    partials = _edge_loss(table, edge_index.reshape(-1))
    return jnp.sum(partials)
